# baseline (speedup 1.0000x reference)
import jax
import jax.numpy as jnp
from jax.experimental import pallas as pl
from jax.experimental.pallas import tpu as pltpu

_EPS = 1e-07
_INF = 100000000.0
_TOPK = 13
_IOU_WEIGHT = 3.0
_RADIUS = 3.0
_LN10 = 2.302585092994046


def _assign_body(psT_ref, pbT_ref, prT_ref, gl_ref, gb_ref, pf_ref,
                 lab_ref, w_ref, box_ref, met_ref):
    v = psT_ref[0]
    pbT = pbT_ref[0]
    prT = prT_ref[...]
    gb = gb_ref[0]
    gl = gl_ref[0]
    pf = pf_ref[0]
    C, N = v.shape
    G = gb.shape[0]

    px1 = pbT[0:1]; py1 = pbT[1:2]; px2 = pbT[2:3]; py2 = pbT[3:4]
    cx = prT[0:1]; cy = prT[1:2]; stride = prT[2:3]
    gx1 = gb[:, 0:1]; gy1 = gb[:, 1:2]; gx2 = gb[:, 2:3]; gy2 = gb[:, 3:4]

    gt_flag = pf > 0.0
    mind = jnp.minimum(jnp.minimum(cx - gx1, cy - gy1),
                       jnp.minimum(gx2 - cx, gy2 - cy))
    in_f = jnp.where(mind > 0.0, pf, 0.0)
    valid = jnp.sum(in_f, axis=0, keepdims=True) > 0.0
    valid_f = jnp.where(valid, 1.0, 0.0)

    gcx = (gx1 + gx2) * 0.5
    gcy = (gy1 + gy2) * 0.5
    dx = cx - gcx
    dy = cy - gcy
    dist = jnp.sqrt(dx * dx + dy * dy) / stride
    dist = dist * valid_f
    soft_center = jnp.exp((dist - _RADIUS) * _LN10)

    ap = (px2 - px1) * (py2 - py1)
    ag = (gx2 - gx1) * (gy2 - gy1)
    ltx = jnp.maximum(px1, gx1); lty = jnp.maximum(py1, gy1)
    rbx = jnp.minimum(px2, gx2); rby = jnp.minimum(py2, gy2)
    ow = jnp.maximum(rbx - ltx, 0.0)
    oh = jnp.maximum(rby - lty, 0.0)
    ov = ow * oh
    iou = ov / jnp.maximum(ap + ag - ov, 1e-06)
    iou_cost = -jnp.log(iou + _EPS) * _IOU_WEIGHT

    e = jnp.exp(-jnp.abs(v))
    sp = jnp.maximum(v, 0.0) + jnp.log1p(e)
    sig = jnp.where(v >= 0.0, 1.0, e) / (1.0 + e)
    t_all = sp * sig * sig

    ones_row = jnp.ones((1, C), jnp.float32)
    s_all = jnp.dot(ones_row, t_all,
                    preferred_element_type=jnp.float32)

    cio = jax.lax.broadcasted_iota(jnp.int32, (G, C), 1)
    onehot = jnp.where(cio == gl, 1.0, 0.0)
    v_sel = jnp.dot(onehot, v, preferred_element_type=jnp.float32)
    sp_sel = jnp.dot(onehot, sp, preferred_element_type=jnp.float32)
    sig_sel = jnp.dot(onehot, sig, preferred_element_type=jnp.float32)
    t_sel = sp_sel * sig_sel * sig_sel
    dlt = iou - sig_sel
    cls_cost = s_all - t_sel + (sp_sel - v_sel * iou) * dlt * dlt

    cost = cls_cost + iou_cost + soft_center
    cost = jnp.where(valid, cost, _INF)

    niota = jax.lax.broadcasted_iota(jnp.int32, (G, N), 1)
    w = iou
    acc = jnp.zeros((G, 1), jnp.float32)
    for _ in range(_TOPK):
        mv = jnp.max(w, axis=1, keepdims=True)
        idx = jnp.min(jnp.where(w == mv, niota, N), axis=1, keepdims=True)
        w = jnp.where(niota == idx, -1.0, w)
        acc = acc + mv
    ks = jnp.maximum(acc.astype(jnp.int32), 1)

    w2 = cost
    matching = jnp.zeros((G, N), jnp.bool_)
    for t in range(_TOPK):
        mv = jnp.min(w2, axis=1, keepdims=True)
        idx = jnp.min(jnp.where(w2 == mv, niota, N), axis=1, keepdims=True)
        hit = niota == idx
        matching = matching | (hit & (t < ks))
        w2 = jnp.where(hit, jnp.inf, w2)
    matching = matching & gt_flag

    cnt = jnp.sum(jnp.where(matching, 1, 0), axis=0, keepdims=True)
    giota = jax.lax.broadcasted_iota(jnp.int32, (G, N), 0)
    rmin = jnp.min(cost, axis=0, keepdims=True)
    amin = jnp.min(jnp.where(cost == rmin, giota, G), axis=0, keepdims=True)
    onehot_min = giota == amin
    multi = cnt > 1
    matching = (multi & onehot_min) | (jnp.logical_not(multi) & matching)

    fg = jnp.sum(jnp.where(matching, 1, 0), axis=0, keepdims=True) > 0
    midx = jnp.min(jnp.where(matching, giota, G), axis=0, keepdims=True)
    midx = jnp.where(fg, midx, 0)
    sel = giota == midx
    selm_f = jnp.where(sel & fg, 1.0, 0.0)

    met_ref[0] = jnp.sum(jnp.where(matching, iou, 0.0), axis=0,
                         keepdims=True)
    lab = jnp.sum(jnp.where(sel, gl, 0), axis=0, keepdims=True)
    lab_ref[0] = jnp.where(fg, lab, C)
    w_ref[0] = jnp.ones((1, N), jnp.float32)
    box_ref[0] = jax.lax.dot_general(
        selm_f, gb, (((0,), (0,)), ((), ())),
        precision=jax.lax.Precision.HIGHEST,
        preferred_element_type=jnp.float32)


def kernel(pred_bboxes, pred_scores, priors, gt_labels, gt_bboxes, pad_bbox_flag):
    B, N, C = pred_scores.shape
    G = gt_bboxes.shape[1]
    psT = jnp.transpose(pred_scores, (0, 2, 1))
    pbT = jnp.transpose(pred_bboxes, (0, 2, 1))
    prT = jnp.transpose(priors[:, :3], (1, 0))
    labs, wts, boxes, mets = pl.pallas_call(
        _assign_body,
        grid=(B,),
        in_specs=[
            pl.BlockSpec((1, C, N), lambda b: (b, 0, 0)),
            pl.BlockSpec((1, 4, N), lambda b: (b, 0, 0)),
            pl.BlockSpec((3, N), lambda b: (0, 0)),
            pl.BlockSpec((1, G, 1), lambda b: (b, 0, 0)),
            pl.BlockSpec((1, G, 4), lambda b: (b, 0, 0)),
            pl.BlockSpec((1, G, 1), lambda b: (b, 0, 0)),
        ],
        out_specs=[
            pl.BlockSpec((1, 1, N), lambda b: (b, 0, 0)),
            pl.BlockSpec((1, 1, N), lambda b: (b, 0, 0)),
            pl.BlockSpec((1, N, 4), lambda b: (b, 0, 0)),
            pl.BlockSpec((1, 1, N), lambda b: (b, 0, 0)),
        ],
        out_shape=[
            jax.ShapeDtypeStruct((B, 1, N), jnp.int32),
            jax.ShapeDtypeStruct((B, 1, N), jnp.float32),
            jax.ShapeDtypeStruct((B, N, 4), jnp.float32),
            jax.ShapeDtypeStruct((B, 1, N), jnp.float32),
        ],
        compiler_params=pltpu.CompilerParams(
            dimension_semantics=("parallel",),
        ),
    )(psT, pbT, prT, gt_labels, gt_bboxes, pad_bbox_flag)
    return labs[:, 0, :], wts[:, 0, :], boxes, mets[:, 0, :]


# speedup vs baseline: 1.3240x; 1.3240x over previous
import jax
import jax.numpy as jnp
from jax.experimental import pallas as pl
from jax.experimental.pallas import tpu as pltpu

_EPS = 1e-07
_INF = 100000000.0
_TOPK = 13
_IOU_WEIGHT = 3.0
_RADIUS = 3.0
_LN10 = 2.302585092994046


def _assign_body(psT_ref, pbT_ref, prT_ref, gl_ref, gb_ref, pf_ref,
                 lab_ref, w_ref, box_ref, met_ref):
    v = psT_ref[0]
    pbT = pbT_ref[0]
    prT = prT_ref[...]
    gb = gb_ref[0]
    gl = gl_ref[0]
    pf = pf_ref[0]
    C, N = v.shape
    G = gb.shape[0]

    px1 = pbT[0:1]; py1 = pbT[1:2]; px2 = pbT[2:3]; py2 = pbT[3:4]
    cx = prT[0:1]; cy = prT[1:2]; stride = prT[2:3]
    gx1 = gb[:, 0:1]; gy1 = gb[:, 1:2]; gx2 = gb[:, 2:3]; gy2 = gb[:, 3:4]

    gt_flag = pf > 0.0
    mind = jnp.minimum(jnp.minimum(cx - gx1, cy - gy1),
                       jnp.minimum(gx2 - cx, gy2 - cy))
    in_f = jnp.where(mind > 0.0, pf, 0.0)
    valid = jnp.sum(in_f, axis=0, keepdims=True) > 0.0
    valid_f = jnp.where(valid, 1.0, 0.0)

    gcx = (gx1 + gx2) * 0.5
    gcy = (gy1 + gy2) * 0.5
    dx = cx - gcx
    dy = cy - gcy
    dist = jnp.sqrt(dx * dx + dy * dy) / stride
    dist = dist * valid_f
    soft_center = jnp.exp((dist - _RADIUS) * _LN10)

    ap = (px2 - px1) * (py2 - py1)
    ag = (gx2 - gx1) * (gy2 - gy1)
    ltx = jnp.maximum(px1, gx1); lty = jnp.maximum(py1, gy1)
    rbx = jnp.minimum(px2, gx2); rby = jnp.minimum(py2, gy2)
    ow = jnp.maximum(rbx - ltx, 0.0)
    oh = jnp.maximum(rby - lty, 0.0)
    ov = ow * oh
    iou = ov / jnp.maximum(ap + ag - ov, 1e-06)
    iou_cost = -jnp.log(iou + _EPS) * _IOU_WEIGHT

    e = jnp.exp(-jnp.abs(v))
    sp = jnp.maximum(v, 0.0) + jnp.log1p(e)
    sig = jnp.where(v >= 0.0, 1.0, e) / (1.0 + e)
    t_all = sp * sig * sig

    ones_row = jnp.ones((1, C), jnp.float32)
    s_all = jnp.dot(ones_row, t_all,
                    preferred_element_type=jnp.float32)

    cio = jax.lax.broadcasted_iota(jnp.int32, (G, C), 1)
    onehot = jnp.where(cio == gl, 1.0, 0.0)
    v_sel = jnp.dot(onehot, v, preferred_element_type=jnp.float32)
    sp_sel = jnp.dot(onehot, sp, preferred_element_type=jnp.float32)
    sig_sel = jnp.dot(onehot, sig, preferred_element_type=jnp.float32)
    t_sel = sp_sel * sig_sel * sig_sel
    dlt = iou - sig_sel
    cls_cost = s_all - t_sel + (sp_sel - v_sel * iou) * dlt * dlt

    cost = cls_cost + iou_cost + soft_center
    cost = jnp.where(valid, cost, _INF)

    niota = jax.lax.broadcasted_iota(jnp.int32, (G, N), 1)
    w = iou
    acc = jnp.zeros((G, 1), jnp.float32)
    for _ in range(_TOPK):
        mv = jnp.max(w, axis=1, keepdims=True)
        idx = jnp.min(jnp.where(w == mv, niota, N), axis=1, keepdims=True)
        w = jnp.where(niota == idx, -1.0, w)
        acc = acc + mv
    ks = jnp.maximum(acc.astype(jnp.int32), 1)

    w2 = cost
    matching = jnp.zeros((G, N), jnp.bool_)
    for t in range(_TOPK):
        mv = jnp.min(w2, axis=1, keepdims=True)
        idx = jnp.min(jnp.where(w2 == mv, niota, N), axis=1, keepdims=True)
        hit = niota == idx
        matching = matching | (hit & (t < ks))
        w2 = jnp.where(hit, jnp.inf, w2)
    matching = matching & gt_flag

    cnt = jnp.sum(jnp.where(matching, 1, 0), axis=0, keepdims=True)
    giota = jax.lax.broadcasted_iota(jnp.int32, (G, N), 0)
    rmin = jnp.min(cost, axis=0, keepdims=True)
    amin = jnp.min(jnp.where(cost == rmin, giota, G), axis=0, keepdims=True)
    onehot_min = giota == amin
    multi = cnt > 1
    matching = (multi & onehot_min) | (jnp.logical_not(multi) & matching)

    fg = jnp.sum(jnp.where(matching, 1, 0), axis=0, keepdims=True) > 0
    midx = jnp.min(jnp.where(matching, giota, G), axis=0, keepdims=True)
    midx = jnp.where(fg, midx, 0)
    sel = giota == midx
    selm = sel & fg

    met_ref[0] = jnp.sum(jnp.where(matching, iou, 0.0), axis=0,
                         keepdims=True)
    lab = jnp.sum(jnp.where(sel, gl, 0), axis=0, keepdims=True)
    lab_ref[0] = jnp.where(fg, lab, C)
    w_ref[0] = jnp.ones((1, N), jnp.float32)
    box_ref[0, 0:1, :] = jnp.sum(jnp.where(selm, gx1, 0.0), axis=0, keepdims=True)
    box_ref[0, 1:2, :] = jnp.sum(jnp.where(selm, gy1, 0.0), axis=0, keepdims=True)
    box_ref[0, 2:3, :] = jnp.sum(jnp.where(selm, gx2, 0.0), axis=0, keepdims=True)
    box_ref[0, 3:4, :] = jnp.sum(jnp.where(selm, gy2, 0.0), axis=0, keepdims=True)


def kernel(pred_bboxes, pred_scores, priors, gt_labels, gt_bboxes, pad_bbox_flag):
    B, N, C = pred_scores.shape
    G = gt_bboxes.shape[1]
    psT = jnp.transpose(pred_scores, (0, 2, 1))
    pbT = jnp.transpose(pred_bboxes, (0, 2, 1))
    prT = jnp.transpose(priors[:, :3], (1, 0))
    labs, wts, boxes, mets = pl.pallas_call(
        _assign_body,
        grid=(B,),
        in_specs=[
            pl.BlockSpec((1, C, N), lambda b: (b, 0, 0)),
            pl.BlockSpec((1, 4, N), lambda b: (b, 0, 0)),
            pl.BlockSpec((3, N), lambda b: (0, 0)),
            pl.BlockSpec((1, G, 1), lambda b: (b, 0, 0)),
            pl.BlockSpec((1, G, 4), lambda b: (b, 0, 0)),
            pl.BlockSpec((1, G, 1), lambda b: (b, 0, 0)),
        ],
        out_specs=[
            pl.BlockSpec((1, 1, N), lambda b: (b, 0, 0)),
            pl.BlockSpec((1, 1, N), lambda b: (b, 0, 0)),
            pl.BlockSpec((1, 4, N), lambda b: (b, 0, 0)),
            pl.BlockSpec((1, 1, N), lambda b: (b, 0, 0)),
        ],
        out_shape=[
            jax.ShapeDtypeStruct((B, 1, N), jnp.int32),
            jax.ShapeDtypeStruct((B, 1, N), jnp.float32),
            jax.ShapeDtypeStruct((B, 4, N), jnp.float32),
            jax.ShapeDtypeStruct((B, 1, N), jnp.float32),
        ],
        compiler_params=pltpu.CompilerParams(
            dimension_semantics=("parallel",),
        ),
    )(psT, pbT, prT, gt_labels, gt_bboxes, pad_bbox_flag)
    return (labs[:, 0, :], wts[:, 0, :],
            jnp.transpose(boxes, (0, 2, 1)), mets[:, 0, :])
